# baseline (speedup 1.0000x reference)
import functools

import jax
import jax.numpy as jnp
from jax import lax
from jax.experimental import pallas as pl
from jax.experimental.pallas import tpu as pltpu
from jax.experimental.pallas import tpu_sc as plsc

_LANES = 16
_CHUNK = 128


def _make_sc_refs(B, K, N, NC, NS):
    NW = NC * NS
    bpw = B // NW
    mesh = plsc.VectorSubcoreMesh(core_axis_name="c", subcore_axis_name="s")

    @functools.partial(
        pl.kernel,
        mesh=mesh,
        compiler_params=pltpu.CompilerParams(use_tc_tiling_on_sc=False),
        out_type=jax.ShapeDtypeStruct((K, B), jnp.int32),
        scratch_types=[
            pltpu.VMEM((bpw,), jnp.int32),
            pltpu.VMEM((K * bpw,), jnp.int32),
            pltpu.VMEM((K * bpw,), jnp.int32),
            pltpu.SemaphoreType.DMA,
        ],
    )
    def sc_refs(idx_hbm, nnsf_hbm, refs_out, idx_v, fidx_v, refs_v, sem):
        wid = lax.axis_index("s") * NC + lax.axis_index("c")
        base = wid * bpw
        pltpu.sync_copy(idx_hbm.at[pl.ds(base, bpw)], idx_v)

        def ext(i, _):
            v = idx_v[pl.ds(i * _LANES, _LANES)]
            for k in range(K):
                fidx_v[pl.ds(k * bpw + i * _LANES, _LANES)] = v + (k * N)
            return 0
        lax.fori_loop(0, bpw // _LANES, ext, 0)

        descs = []
        for j in range(K * bpw // _CHUNK):
            sl = pl.ds(j * _CHUNK, _CHUNK)
            descs.append(
                pltpu.async_copy(nnsf_hbm.at[fidx_v.at[sl]], refs_v.at[sl], sem))
        for d in descs:
            d.wait()
        for k in range(K):
            pltpu.sync_copy(refs_v.at[pl.ds(k * bpw, bpw)],
                            refs_out.at[k, pl.ds(base, bpw)])

    return sc_refs


def _make_sc_gather(B, K, D, NC, NS):
    NW = NC * NS
    bpw = B // NW
    mesh = plsc.VectorSubcoreMesh(core_axis_name="c", subcore_axis_name="s")

    @functools.partial(
        pl.kernel,
        mesh=mesh,
        compiler_params=pltpu.CompilerParams(use_tc_tiling_on_sc=False),
        out_type=jax.ShapeDtypeStruct((K, B, D), jnp.float32),
        scratch_types=[
            pltpu.VMEM((K * bpw,), jnp.int32),
            pltpu.VMEM((2, bpw, D), jnp.float32),
            pltpu.SemaphoreType.DMA,
            pltpu.SemaphoreType.DMA,
        ],
    )
    def sc_gather(refs_hbm, t_hbm, tg_out, refs_v, tgb, gsem, osem):
        wid = lax.axis_index("s") * NC + lax.axis_index("c")
        base = wid * bpw
        for k in range(K):
            pltpu.sync_copy(refs_hbm.at[k, pl.ds(base, bpw)],
                            refs_v.at[pl.ds(k * bpw, bpw)])
        outd = [None, None]
        for k in range(K):
            b = k & 1
            if outd[b] is not None:
                outd[b].wait()
            descs = []
            for j in range(bpw // _CHUNK):
                descs.append(pltpu.async_copy(
                    t_hbm.at[refs_v.at[pl.ds(k * bpw + j * _CHUNK, _CHUNK)]],
                    tgb.at[b, pl.ds(j * _CHUNK, _CHUNK)], gsem))
            for d in descs:
                d.wait()
            outd[b] = pltpu.async_copy(tgb.at[b],
                                       tg_out.at[k, pl.ds(base, bpw)], osem)
        outd[0].wait()
        outd[1].wait()

    return sc_gather


def _table_body(HID, tdT_ref, memT_ref, wp_ref, wa_ref, ba_ref, t_ref):
    pos = lax.dot_general(tdT_ref[...], wp_ref[...],
                          (((0,), (0,)), ((), ())),
                          preferred_element_type=jnp.float32)
    mema = lax.dot_general(memT_ref[...], wa_ref[...],
                           (((0,), (0,)), ((), ())),
                           preferred_element_type=jnp.float32) + ba_ref[...]
    t_ref[:, :HID] = pos
    t_ref[:, HID:] = mema


def _combine_body(K, HID, dT_ref, tg_ref, wp_ref, wf_ref, bf_ref, out_ref):
    dT = dT_ref[...]
    pos_q = lax.dot_general(dT, wp_ref[...], (((0,), (0,)), ((), ())),
                            preferred_element_type=jnp.float32)
    field = lax.dot_general(dT, wf_ref[...], (((0,), (0,)), ((), ())),
                            preferred_element_type=jnp.float32) + bf_ref[...]
    diffs, negs = [], []
    for k in range(K):
        d = pos_q - tg_ref[k, :, :HID]
        diffs.append(d)
        negs.append(-jnp.sqrt(jnp.sum(d * d, axis=1, keepdims=True)))
    m = negs[0]
    for k in range(1, K):
        m = jnp.maximum(m, negs[k])
    es = [jnp.exp(n - m) for n in negs]
    inv = 1.0 / sum(es)
    wdiff = sum(es[k] * diffs[k] for k in range(K)) * inv
    out = sum(es[k] * tg_ref[k, :, HID:] for k in range(K)) * inv
    nc = out.shape[1]
    for h in range(HID):
        out += wdiff[:, h:h + 1] * field[:, h * nc:(h + 1) * nc]
    out_ref[...] = out


def kernel(idx, date, nns, train_dates, mem, W_pos, b_pos, W_field, b_field,
           W_adapt, b_adapt):
    B = idx.shape[0]
    K = nns.shape[1]
    N, F = train_dates.shape
    C = mem.shape[1]
    HID = W_pos.shape[1]
    D = HID + C

    tdT = train_dates.T
    memT = mem.T
    nnsf = nns.astype(jnp.int32).T.reshape(-1)
    dateT = date.T

    info = plsc.get_sparse_core_info()
    NC, NS = info.num_cores, info.num_subcores

    refs = _make_sc_refs(B, K, N, NC, NS)(idx.astype(jnp.int32), nnsf)

    NB = 4096
    tparams = pltpu.CompilerParams(fuse_transposed_lhs_in_matmul=True)
    table = pl.pallas_call(
        functools.partial(_table_body, HID),
        grid=(pl.cdiv(N, NB),),
        in_specs=[
            pl.BlockSpec((F, NB), lambda i: (0, i)),
            pl.BlockSpec((C, NB), lambda i: (0, i)),
            pl.BlockSpec((F, HID), lambda i: (0, 0)),
            pl.BlockSpec((C, C), lambda i: (0, 0)),
            pl.BlockSpec((1, C), lambda i: (0, 0)),
        ],
        out_specs=pl.BlockSpec((NB, D), lambda i: (i, 0)),
        out_shape=jax.ShapeDtypeStruct((N, D), jnp.float32),
        compiler_params=tparams,
    )(tdT, memT, W_pos, W_adapt, b_adapt.reshape(1, C))

    tg = _make_sc_gather(B, K, D, NC, NS)(refs, table)

    BB = 1024
    out = pl.pallas_call(
        functools.partial(_combine_body, K, HID),
        grid=(B // BB,),
        in_specs=[
            pl.BlockSpec((F, BB), lambda i: (0, i)),
            pl.BlockSpec((K, BB, D), lambda i: (0, i, 0)),
            pl.BlockSpec((F, HID), lambda i: (0, 0)),
            pl.BlockSpec((F, HID * C), lambda i: (0, 0)),
            pl.BlockSpec((1, HID * C), lambda i: (0, 0)),
        ],
        out_specs=pl.BlockSpec((BB, C), lambda i: (i, 0)),
        out_shape=jax.ShapeDtypeStruct((B, C), jnp.float32),
        compiler_params=tparams,
    )(dateT, tg, W_pos, W_field, b_field.reshape(1, HID * C))
    return out


# speedup vs baseline: 1.1010x; 1.1010x over previous
import functools

import jax
import jax.numpy as jnp
from jax import lax
from jax.experimental import pallas as pl
from jax.experimental.pallas import tpu as pltpu
from jax.experimental.pallas import tpu_sc as plsc

_LANES = 16
_CHUNK = 128


def _make_sc_refs(B, K, N, NC, NS):
    NW = NC * NS
    bpw = B // NW
    mesh = plsc.VectorSubcoreMesh(core_axis_name="c", subcore_axis_name="s")

    @functools.partial(
        pl.kernel,
        mesh=mesh,
        compiler_params=pltpu.CompilerParams(use_tc_tiling_on_sc=False),
        out_type=jax.ShapeDtypeStruct((K, B), jnp.int32),
        scratch_types=[
            pltpu.VMEM((bpw,), jnp.int32),
            pltpu.VMEM((K * bpw,), jnp.int32),
            pltpu.VMEM((K * bpw,), jnp.int32),
            pltpu.SemaphoreType.DMA,
        ],
    )
    def sc_refs(idx_hbm, nnsf_hbm, refs_out, idx_v, fidx_v, refs_v, sem):
        wid = lax.axis_index("s") * NC + lax.axis_index("c")
        base = wid * bpw
        pltpu.sync_copy(idx_hbm.at[pl.ds(base, bpw)], idx_v)

        def ext(i, _):
            v = idx_v[pl.ds(i * _LANES, _LANES)]
            for k in range(K):
                fidx_v[pl.ds(k * bpw + i * _LANES, _LANES)] = v + (k * N)
            return 0
        lax.fori_loop(0, bpw // _LANES, ext, 0)

        descs = []
        for j in range(K * bpw // _CHUNK):
            sl = pl.ds(j * _CHUNK, _CHUNK)
            descs.append(
                pltpu.async_copy(nnsf_hbm.at[fidx_v.at[sl]], refs_v.at[sl], sem))
        for d in descs:
            d.wait()
        for k in range(K):
            pltpu.sync_copy(refs_v.at[pl.ds(k * bpw, bpw)],
                            refs_out.at[k, pl.ds(base, bpw)])

    return sc_refs


def _make_sc_gather(B, K, D, NC, NS):
    NW = NC * NS
    bpw = B // NW
    mesh = plsc.VectorSubcoreMesh(core_axis_name="c", subcore_axis_name="s")

    @functools.partial(
        pl.kernel,
        mesh=mesh,
        compiler_params=pltpu.CompilerParams(use_tc_tiling_on_sc=False),
        out_type=jax.ShapeDtypeStruct((K, B, D), jnp.float32),
        scratch_types=[
            pltpu.VMEM((K * bpw,), jnp.int32),
            pltpu.VMEM((2, bpw, D), jnp.float32),
            pltpu.SemaphoreType.DMA,
            pltpu.SemaphoreType.DMA,
        ],
    )
    def sc_gather(refs_hbm, t_hbm, tg_out, refs_v, tgb, gsem, osem):
        wid = lax.axis_index("s") * NC + lax.axis_index("c")
        base = wid * bpw
        for k in range(K):
            pltpu.sync_copy(refs_hbm.at[k, pl.ds(base, bpw)],
                            refs_v.at[pl.ds(k * bpw, bpw)])
        outd = [None, None]
        for k in range(K):
            b = k & 1
            if outd[b] is not None:
                outd[b].wait()
            descs = []
            for j in range(bpw // _CHUNK):
                descs.append(pltpu.async_copy(
                    t_hbm.at[refs_v.at[pl.ds(k * bpw + j * _CHUNK, _CHUNK)]],
                    tgb.at[b, pl.ds(j * _CHUNK, _CHUNK)], gsem))
            for d in descs:
                d.wait()
            outd[b] = pltpu.async_copy(tgb.at[b],
                                       tg_out.at[k, pl.ds(base, bpw)], osem)
        outd[0].wait()
        outd[1].wait()

    return sc_gather


def _table_body(tdT_ref, memT_ref, wpe_ref, wae_ref, bae_ref, t_ref):
    pos = lax.dot_general(tdT_ref[...], wpe_ref[...],
                          (((0,), (0,)), ((), ())),
                          preferred_element_type=jnp.float32)
    mema = lax.dot_general(memT_ref[...], wae_ref[...],
                           (((0,), (0,)), ((), ())),
                           preferred_element_type=jnp.float32)
    t_ref[...] = pos + mema + bae_ref[...]


def _seg_matrices(G, S):
    jj = lax.broadcasted_iota(jnp.int32, (G, G * S), 1)
    gg = lax.broadcasted_iota(jnp.int32, (G, G * S), 0)
    bcast = (jj // S == gg).astype(jnp.float32)
    jj2 = lax.broadcasted_iota(jnp.int32, (G * S, G), 0)
    gg2 = lax.broadcasted_iota(jnp.int32, (G * S, G), 1)
    gsum = (jj2 // S == gg2).astype(jnp.float32)
    jj3 = lax.broadcasted_iota(jnp.int32, (G * S, S), 0)
    ss3 = lax.broadcasted_iota(jnp.int32, (G * S, S), 1)
    psum = (jj3 % S == ss3).astype(jnp.float32)
    return bcast, gsum, psum


def _mm(a, b, precision=None):
    return lax.dot_general(a, b, (((1,), (0,)), ((), ())),
                           preferred_element_type=jnp.float32,
                           precision=precision)


def _combine_body(K, HID, C, dT_ref, tg_ref, wp_ref, wf_ref, bf_ref, out_ref):
    dT = dT_ref[...]
    pos_q = lax.dot_general(dT, wp_ref[...], (((0,), (0,)), ((), ())),
                            preferred_element_type=jnp.float32)
    field = lax.dot_general(dT, wf_ref[...], (((0,), (0,)), ((), ())),
                            preferred_element_type=jnp.float32) + bf_ref[...]
    diffs = [pos_q - tg_ref[k, :, :HID] for k in range(K)]
    diff_cat = jnp.concatenate(diffs, axis=1)

    kb, kg, _ = _seg_matrices(K, HID)
    sq = _mm(diff_cat * diff_cat, kg)
    neg = -jnp.sqrt(sq)
    m = jnp.max(neg, axis=1, keepdims=True)
    es = jnp.exp(neg - m)
    dist = es / jnp.sum(es, axis=1, keepdims=True)
    distb = _mm(dist, kb)
    wdiff = sum(distb[:, k * HID:(k + 1) * HID] * diffs[k] for k in range(K))
    wmem = sum(distb[:, k * HID:(k + 1) * HID] * tg_ref[k, :, HID:]
               for k in range(K))

    hb, _, hp = _seg_matrices(HID, C)
    corr = _mm(_mm(wdiff, hb) * field, hp)
    out_ref[...] = wmem + corr


def kernel(idx, date, nns, train_dates, mem, W_pos, b_pos, W_field, b_field,
           W_adapt, b_adapt):
    B = idx.shape[0]
    K = nns.shape[1]
    N, F = train_dates.shape
    C = mem.shape[1]
    HID = W_pos.shape[1]
    D = HID + C

    tdT = train_dates.T
    memT = mem.T
    nnsf = nns.astype(jnp.int32).T.reshape(-1)
    dateT = date.T

    info = plsc.get_sparse_core_info()
    NC, NS = info.num_cores, info.num_subcores

    refs = _make_sc_refs(B, K, N, NC, NS)(idx.astype(jnp.int32), nnsf)

    wpe = jnp.concatenate([W_pos, jnp.zeros((F, C), jnp.float32)], axis=1)
    wae = jnp.concatenate([jnp.zeros((C, HID), jnp.float32), W_adapt], axis=1)
    bae = jnp.concatenate([jnp.zeros((HID,), jnp.float32), b_adapt]).reshape(1, D)

    NB = 4096
    tparams = pltpu.CompilerParams(fuse_transposed_lhs_in_matmul=True)
    table = pl.pallas_call(
        _table_body,
        grid=(pl.cdiv(N, NB),),
        in_specs=[
            pl.BlockSpec((F, NB), lambda i: (0, i)),
            pl.BlockSpec((C, NB), lambda i: (0, i)),
            pl.BlockSpec((F, D), lambda i: (0, 0)),
            pl.BlockSpec((C, D), lambda i: (0, 0)),
            pl.BlockSpec((1, D), lambda i: (0, 0)),
        ],
        out_specs=pl.BlockSpec((NB, D), lambda i: (i, 0)),
        out_shape=jax.ShapeDtypeStruct((N, D), jnp.float32),
        compiler_params=tparams,
    )(tdT, memT, wpe, wae, bae)

    tg = _make_sc_gather(B, K, D, NC, NS)(refs, table)

    BB = 1024
    out = pl.pallas_call(
        functools.partial(_combine_body, K, HID, C),
        grid=(B // BB,),
        in_specs=[
            pl.BlockSpec((F, BB), lambda i: (0, i)),
            pl.BlockSpec((K, BB, D), lambda i: (0, i, 0)),
            pl.BlockSpec((F, HID), lambda i: (0, 0)),
            pl.BlockSpec((F, HID * C), lambda i: (0, 0)),
            pl.BlockSpec((1, HID * C), lambda i: (0, 0)),
        ],
        out_specs=pl.BlockSpec((BB, C), lambda i: (i, 0)),
        out_shape=jax.ShapeDtypeStruct((B, C), jnp.float32),
        compiler_params=tparams,
    )(dateT, tg, W_pos, W_field, b_field.reshape(1, HID * C))
    return out


# speedup vs baseline: 1.6643x; 1.5116x over previous
import functools

import jax
import jax.numpy as jnp
from jax import lax
from jax.experimental import pallas as pl
from jax.experimental.pallas import tpu as pltpu
from jax.experimental.pallas import tpu_sc as plsc

_LANES = 16
_CHUNK = 128


def _split_body(K, nnsT_ref, *out_refs):
    for k in range(K):
        out_refs[k][...] = nnsT_ref[k, :]


def _make_sc_refs(B, K, NC, NS):
    NW = NC * NS
    bpw = B // NW
    mesh = plsc.VectorSubcoreMesh(core_axis_name="c", subcore_axis_name="s")

    @functools.partial(
        pl.kernel,
        mesh=mesh,
        compiler_params=pltpu.CompilerParams(use_tc_tiling_on_sc=False),
        out_type=jax.ShapeDtypeStruct((K, B), jnp.int32),
        scratch_types=[
            pltpu.VMEM((bpw,), jnp.int32),
            pltpu.VMEM((K * bpw,), jnp.int32),
            pltpu.SemaphoreType.DMA,
        ],
    )
    def sc_refs(idx_hbm, *args):
        nn_hbms = args[:K]
        refs_out = args[K]
        idx_v, refs_v, sem = args[K + 1:]
        wid = lax.axis_index("s") * NC + lax.axis_index("c")
        base = wid * bpw
        pltpu.sync_copy(idx_hbm.at[pl.ds(base, bpw)], idx_v)
        descs = []
        for k in range(K):
            for j in range(bpw // _CHUNK):
                isl = pl.ds(j * _CHUNK, _CHUNK)
                osl = pl.ds(k * bpw + j * _CHUNK, _CHUNK)
                descs.append(pltpu.async_copy(
                    nn_hbms[k].at[idx_v.at[isl]], refs_v.at[osl], sem))
        for d in descs:
            d.wait()
        for k in range(K):
            pltpu.sync_copy(refs_v.at[pl.ds(k * bpw, bpw)],
                            refs_out.at[k, pl.ds(base, bpw)])

    return sc_refs


def _make_sc_gather(B, K, D, NC, NS):
    NW = NC * NS
    bpw = B // NW
    mesh = plsc.VectorSubcoreMesh(core_axis_name="c", subcore_axis_name="s")

    @functools.partial(
        pl.kernel,
        mesh=mesh,
        compiler_params=pltpu.CompilerParams(use_tc_tiling_on_sc=False),
        out_type=jax.ShapeDtypeStruct((K, B, D), jnp.float32),
        scratch_types=[
            pltpu.VMEM((K * bpw,), jnp.int32),
            pltpu.VMEM((2, bpw, D), jnp.float32),
            pltpu.SemaphoreType.DMA,
            pltpu.SemaphoreType.DMA,
        ],
    )
    def sc_gather(refs_hbm, t_hbm, tg_out, refs_v, tgb, gsem, osem):
        wid = lax.axis_index("s") * NC + lax.axis_index("c")
        base = wid * bpw
        for k in range(K):
            pltpu.sync_copy(refs_hbm.at[k, pl.ds(base, bpw)],
                            refs_v.at[pl.ds(k * bpw, bpw)])
        outd = [None, None]
        for k in range(K):
            b = k & 1
            if outd[b] is not None:
                outd[b].wait()
            descs = []
            for j in range(bpw // _CHUNK):
                descs.append(pltpu.async_copy(
                    t_hbm.at[refs_v.at[pl.ds(k * bpw + j * _CHUNK, _CHUNK)]],
                    tgb.at[b, pl.ds(j * _CHUNK, _CHUNK)], gsem))
            for d in descs:
                d.wait()
            outd[b] = pltpu.async_copy(tgb.at[b],
                                       tg_out.at[k, pl.ds(base, bpw)], osem)
        outd[0].wait()
        outd[1].wait()

    return sc_gather


def _table_body(tdT_ref, memT_ref, wpe_ref, wae_ref, bae_ref, t_ref):
    pos = lax.dot_general(tdT_ref[...], wpe_ref[...],
                          (((0,), (0,)), ((), ())),
                          preferred_element_type=jnp.float32)
    mema = lax.dot_general(memT_ref[...], wae_ref[...],
                           (((0,), (0,)), ((), ())),
                           preferred_element_type=jnp.float32)
    t_ref[...] = pos + mema + bae_ref[...]


def _seg_matrices(G, S):
    jj = lax.broadcasted_iota(jnp.int32, (G, G * S), 1)
    gg = lax.broadcasted_iota(jnp.int32, (G, G * S), 0)
    bcast = (jj // S == gg).astype(jnp.float32)
    jj2 = lax.broadcasted_iota(jnp.int32, (G * S, G), 0)
    gg2 = lax.broadcasted_iota(jnp.int32, (G * S, G), 1)
    gsum = (jj2 // S == gg2).astype(jnp.float32)
    jj3 = lax.broadcasted_iota(jnp.int32, (G * S, S), 0)
    ss3 = lax.broadcasted_iota(jnp.int32, (G * S, S), 1)
    psum = (jj3 % S == ss3).astype(jnp.float32)
    return bcast, gsum, psum


def _mm(a, b, precision=None):
    return lax.dot_general(a, b, (((1,), (0,)), ((), ())),
                           preferred_element_type=jnp.float32,
                           precision=precision)


def _combine_body(K, HID, C, dT_ref, tg_ref, wp_ref, wf_ref, bf_ref, out_ref):
    dT = dT_ref[...]
    pos_q = lax.dot_general(dT, wp_ref[...], (((0,), (0,)), ((), ())),
                            preferred_element_type=jnp.float32)
    field = lax.dot_general(dT, wf_ref[...], (((0,), (0,)), ((), ())),
                            preferred_element_type=jnp.float32) + bf_ref[...]
    diffs = [pos_q - tg_ref[k, :, :HID] for k in range(K)]
    diff_cat = jnp.concatenate(diffs, axis=1)

    kb, kg, _ = _seg_matrices(K, HID)
    sq = _mm(diff_cat * diff_cat, kg)
    neg = -jnp.sqrt(sq)
    m = jnp.max(neg, axis=1, keepdims=True)
    es = jnp.exp(neg - m)
    dist = es / jnp.sum(es, axis=1, keepdims=True)
    distb = _mm(dist, kb)
    wdiff = sum(distb[:, k * HID:(k + 1) * HID] * diffs[k] for k in range(K))
    wmem = sum(distb[:, k * HID:(k + 1) * HID] * tg_ref[k, :, HID:]
               for k in range(K))

    hb, _, hp = _seg_matrices(HID, C)
    corr = _mm(_mm(wdiff, hb) * field, hp)
    out_ref[...] = wmem + corr


def kernel(idx, date, nns, train_dates, mem, W_pos, b_pos, W_field, b_field,
           W_adapt, b_adapt):
    B = idx.shape[0]
    K = nns.shape[1]
    N, F = train_dates.shape
    C = mem.shape[1]
    HID = W_pos.shape[1]
    D = HID + C

    tdT = train_dates.T
    memT = mem.T
    nnsT = nns.astype(jnp.int32).T
    dateT = date.T

    info = plsc.get_sparse_core_info()
    NC, NS = info.num_cores, info.num_subcores

    SB = 8192
    nn_cols = pl.pallas_call(
        functools.partial(_split_body, K),
        grid=(pl.cdiv(N, SB),),
        in_specs=[pl.BlockSpec((K, SB), lambda i: (0, i))],
        out_specs=[pl.BlockSpec((SB,), lambda i: (i,)) for _ in range(K)],
        out_shape=[jax.ShapeDtypeStruct((N,), jnp.int32) for _ in range(K)],
    )(nnsT)

    refs = _make_sc_refs(B, K, NC, NS)(idx.astype(jnp.int32), *nn_cols)

    wpe = jnp.concatenate([W_pos, jnp.zeros((F, C), jnp.float32)], axis=1)
    wae = jnp.concatenate([jnp.zeros((C, HID), jnp.float32), W_adapt], axis=1)
    bae = jnp.concatenate([jnp.zeros((HID,), jnp.float32), b_adapt]).reshape(1, D)

    NB = 4096
    tparams = pltpu.CompilerParams(fuse_transposed_lhs_in_matmul=True)
    table = pl.pallas_call(
        _table_body,
        grid=(pl.cdiv(N, NB),),
        in_specs=[
            pl.BlockSpec((F, NB), lambda i: (0, i)),
            pl.BlockSpec((C, NB), lambda i: (0, i)),
            pl.BlockSpec((F, D), lambda i: (0, 0)),
            pl.BlockSpec((C, D), lambda i: (0, 0)),
            pl.BlockSpec((1, D), lambda i: (0, 0)),
        ],
        out_specs=pl.BlockSpec((NB, D), lambda i: (i, 0)),
        out_shape=jax.ShapeDtypeStruct((N, D), jnp.float32),
        compiler_params=tparams,
    )(tdT, memT, wpe, wae, bae)

    tg = _make_sc_gather(B, K, D, NC, NS)(refs, table)

    BB = 1024
    out = pl.pallas_call(
        functools.partial(_combine_body, K, HID, C),
        grid=(B // BB,),
        in_specs=[
            pl.BlockSpec((F, BB), lambda i: (0, i)),
            pl.BlockSpec((K, BB, D), lambda i: (0, i, 0)),
            pl.BlockSpec((F, HID), lambda i: (0, 0)),
            pl.BlockSpec((F, HID * C), lambda i: (0, 0)),
            pl.BlockSpec((1, HID * C), lambda i: (0, 0)),
        ],
        out_specs=pl.BlockSpec((BB, C), lambda i: (i, 0)),
        out_shape=jax.ShapeDtypeStruct((B, C), jnp.float32),
        compiler_params=tparams,
    )(dateT, tg, W_pos, W_field, b_field.reshape(1, HID * C))
    return out


# speedup vs baseline: 1.7434x; 1.0475x over previous
import functools

import jax
import jax.numpy as jnp
from jax import lax
from jax.experimental import pallas as pl
from jax.experimental.pallas import tpu as pltpu
from jax.experimental.pallas import tpu_sc as plsc

_LANES = 16
_CHUNK = 128


def _split_body(K, nnsT_ref, *out_refs):
    for k in range(K):
        out_refs[k][...] = nnsT_ref[k, :]


def _make_sc_refs(B, K, NC, NS):
    NW = NC * NS
    bpw = B // NW
    mesh = plsc.VectorSubcoreMesh(core_axis_name="c", subcore_axis_name="s")

    @functools.partial(
        pl.kernel,
        mesh=mesh,
        compiler_params=pltpu.CompilerParams(use_tc_tiling_on_sc=False),
        out_type=jax.ShapeDtypeStruct((K, B), jnp.int32),
        scratch_types=[
            pltpu.VMEM((bpw,), jnp.int32),
            pltpu.VMEM((K * bpw,), jnp.int32),
            pltpu.SemaphoreType.DMA,
        ],
    )
    def sc_refs(idx_hbm, *args):
        nn_hbms = args[:K]
        refs_out = args[K]
        idx_v, refs_v, sem = args[K + 1:]
        wid = lax.axis_index("s") * NC + lax.axis_index("c")
        base = wid * bpw
        pltpu.sync_copy(idx_hbm.at[pl.ds(base, bpw)], idx_v)
        descs = []
        for k in range(K):
            for j in range(bpw // _CHUNK):
                isl = pl.ds(j * _CHUNK, _CHUNK)
                osl = pl.ds(k * bpw + j * _CHUNK, _CHUNK)
                descs.append(pltpu.async_copy(
                    nn_hbms[k].at[idx_v.at[isl]], refs_v.at[osl], sem))
        for d in descs:
            d.wait()
        for k in range(K):
            pltpu.sync_copy(refs_v.at[pl.ds(k * bpw, bpw)],
                            refs_out.at[k, pl.ds(base, bpw)])

    return sc_refs


def _make_sc_gather(B, K, D, NC, NS):
    NW = NC * NS
    bpw = B // NW
    mesh = plsc.VectorSubcoreMesh(core_axis_name="c", subcore_axis_name="s")

    @functools.partial(
        pl.kernel,
        mesh=mesh,
        compiler_params=pltpu.CompilerParams(use_tc_tiling_on_sc=False),
        out_type=jax.ShapeDtypeStruct((K, B, D), jnp.float32),
        scratch_types=[
            pltpu.VMEM((K * bpw,), jnp.int32),
            pltpu.VMEM((2, bpw, D), jnp.float32),
            pltpu.SemaphoreType.DMA,
            pltpu.SemaphoreType.DMA,
        ],
    )
    def sc_gather(refs_hbm, t_hbm, tg_out, refs_v, tgb, gsem, osem):
        wid = lax.axis_index("s") * NC + lax.axis_index("c")
        base = wid * bpw
        for k in range(K):
            pltpu.sync_copy(refs_hbm.at[k, pl.ds(base, bpw)],
                            refs_v.at[pl.ds(k * bpw, bpw)])
        outd = [None, None]
        for k in range(K):
            b = k & 1
            if outd[b] is not None:
                outd[b].wait()
            descs = []
            for j in range(bpw // _CHUNK):
                descs.append(pltpu.async_copy(
                    t_hbm.at[refs_v.at[pl.ds(k * bpw + j * _CHUNK, _CHUNK)]],
                    tgb.at[b, pl.ds(j * _CHUNK, _CHUNK)], gsem))
            for d in descs:
                d.wait()
            outd[b] = pltpu.async_copy(tgb.at[b],
                                       tg_out.at[k, pl.ds(base, bpw)], osem)
        outd[0].wait()
        outd[1].wait()

    return sc_gather


def _table_body(tdT_ref, memT_ref, wpe_ref, wae_ref, bae_ref, t_ref):
    pos = lax.dot_general(tdT_ref[...], wpe_ref[...],
                          (((0,), (0,)), ((), ())),
                          preferred_element_type=jnp.float32)
    mema = lax.dot_general(memT_ref[...], wae_ref[...],
                           (((0,), (0,)), ((), ())),
                           preferred_element_type=jnp.float32)
    t_ref[...] = pos + mema + bae_ref[...]


def _mm(a, b, precision=None):
    return lax.dot_general(a, b, (((1,), (0,)), ((), ())),
                           preferred_element_type=jnp.float32,
                           precision=precision)


def _iota2(shape, dim):
    return lax.broadcasted_iota(jnp.int32, shape, dim)


def _combine_body(K, HID, C, FW, d2_ref, tg2_ref, wq_ref, wf2_ref, bf2_ref,
                  out_ref):
    d2 = d2_ref[...]
    pos_q2 = _mm(d2, wq_ref[...])
    field2 = _mm(d2, wf2_ref[...]) + bf2_ref[...]
    diff2s = [pos_q2 - tg2_ref[k] for k in range(K)]
    d2cat = jnp.concatenate(diff2s, axis=1)

    KW = K * 128
    r = _iota2((KW, 2 * K), 0)
    c = _iota2((KW, 2 * K), 1)
    k_, grp = r // 128, (r % 128) // 32
    sqg = (((c < K) & (k_ == c) & (grp == 0)) |
           ((c >= K) & (k_ == c - K) & (grp == 2))).astype(jnp.float32)
    sq16 = _mm(d2cat * d2cat, sqg)
    neg = -jnp.sqrt(sq16)
    m_e = jnp.max(neg[:, :K], axis=1, keepdims=True)
    m_o = jnp.max(neg[:, K:], axis=1, keepdims=True)
    es_e = jnp.exp(neg[:, :K] - m_e)
    es_o = jnp.exp(neg[:, K:] - m_o)
    dist16 = jnp.concatenate(
        [es_e / jnp.sum(es_e, axis=1, keepdims=True),
         es_o / jnp.sum(es_o, axis=1, keepdims=True)], axis=1)

    c2 = _iota2((2 * K, KW), 0)
    r2 = _iota2((2 * K, KW), 1)
    k2, half = r2 // 128, (r2 % 128) // 64
    db = (((c2 < K) & (k2 == c2) & (half == 0)) |
          ((c2 >= K) & (k2 == c2 - K) & (half == 1))).astype(jnp.float32)
    distb2 = _mm(dist16, db)
    s2 = sum(distb2[:, k * 128:(k + 1) * 128] * diff2s[k] for k in range(K))

    h2 = _iota2((128, 2 * FW), 0)
    j2 = _iota2((128, 2 * FW), 1)
    hcol = (j2 % FW) // C
    hb2 = (((j2 < FW) & (h2 == hcol)) |
           ((j2 >= FW) & (h2 == 64 + hcol))).astype(jnp.float32)
    j3 = _iota2((2 * FW, 2 * C), 0)
    c3 = _iota2((2 * FW, 2 * C), 1)
    ps2 = ((c3 == (j3 >= FW) * C + j3 % C)).astype(jnp.float32)
    corr2 = _mm(_mm(s2, hb2) * field2, ps2)

    p4 = _iota2((128, 2 * C), 0)
    c4 = _iota2((128, 2 * C), 1)
    pm = (((c4 < C) & (p4 == c4 + HID)) |
          ((c4 >= C) & (p4 == c4 + 64 + HID - C))).astype(jnp.float32)
    out_ref[...] = corr2 - _mm(s2, pm)


def kernel(idx, date, nns, train_dates, mem, W_pos, b_pos, W_field, b_field,
           W_adapt, b_adapt):
    B = idx.shape[0]
    K = nns.shape[1]
    N, F = train_dates.shape
    C = mem.shape[1]
    HID = W_pos.shape[1]
    D = HID + C

    tdT = train_dates.T
    memT = mem.T
    nnsT = nns.astype(jnp.int32).T
    dateT = date.T

    info = plsc.get_sparse_core_info()
    NC, NS = info.num_cores, info.num_subcores

    SB = 8192
    nn_cols = pl.pallas_call(
        functools.partial(_split_body, K),
        grid=(pl.cdiv(N, SB),),
        in_specs=[pl.BlockSpec((K, SB), lambda i: (0, i))],
        out_specs=[pl.BlockSpec((SB,), lambda i: (i,)) for _ in range(K)],
        out_shape=[jax.ShapeDtypeStruct((N,), jnp.int32) for _ in range(K)],
    )(nnsT)

    refs = _make_sc_refs(B, K, NC, NS)(idx.astype(jnp.int32), *nn_cols)

    wpe = jnp.concatenate([W_pos, jnp.zeros((F, C), jnp.float32)], axis=1)
    wae = jnp.concatenate([jnp.zeros((C, HID), jnp.float32), W_adapt], axis=1)
    bae = jnp.concatenate([jnp.zeros((HID,), jnp.float32), b_adapt]).reshape(1, D)

    NB = 4096
    tparams = pltpu.CompilerParams(fuse_transposed_lhs_in_matmul=True)
    table = pl.pallas_call(
        _table_body,
        grid=(pl.cdiv(N, NB),),
        in_specs=[
            pl.BlockSpec((F, NB), lambda i: (0, i)),
            pl.BlockSpec((C, NB), lambda i: (0, i)),
            pl.BlockSpec((F, D), lambda i: (0, 0)),
            pl.BlockSpec((C, D), lambda i: (0, 0)),
            pl.BlockSpec((1, D), lambda i: (0, 0)),
        ],
        out_specs=pl.BlockSpec((NB, D), lambda i: (i, 0)),
        out_shape=jax.ShapeDtypeStruct((N, D), jnp.float32),
        compiler_params=tparams,
    )(tdT, memT, wpe, wae, bae)

    tg = _make_sc_gather(B, K, D, NC, NS)(refs, table)

    FW = HID * C
    tg2 = tg.reshape(K, B * D // 128, 128)
    date2 = date.reshape(B // 2, 2 * F)
    wq = jnp.zeros((2 * F, 128), jnp.float32)
    wq = wq.at[:F, :HID].set(W_pos).at[F:, 64:64 + HID].set(W_pos)
    wf2 = jnp.zeros((2 * F, 2 * FW), jnp.float32)
    wf2 = wf2.at[:F, :FW].set(W_field).at[F:, FW:].set(W_field)
    bf2 = jnp.concatenate([b_field, b_field]).reshape(1, 2 * FW)

    BB = 1024
    PB = BB // 2
    out2 = pl.pallas_call(
        functools.partial(_combine_body, K, HID, C, FW),
        grid=(B // BB,),
        in_specs=[
            pl.BlockSpec((PB, 2 * F), lambda i: (i, 0)),
            pl.BlockSpec((K, PB, 128), lambda i: (0, i, 0)),
            pl.BlockSpec((2 * F, 128), lambda i: (0, 0)),
            pl.BlockSpec((2 * F, 2 * FW), lambda i: (0, 0)),
            pl.BlockSpec((1, 2 * FW), lambda i: (0, 0)),
        ],
        out_specs=pl.BlockSpec((PB, 2 * C), lambda i: (i, 0)),
        out_shape=jax.ShapeDtypeStruct((B // 2, 2 * C), jnp.float32),
        compiler_params=tparams,
    )(date2, tg2, wq, wf2, bf2)
    return out2.reshape(B, C)


# speedup vs baseline: 2.7262x; 1.5637x over previous
import functools

import jax
import jax.numpy as jnp
from jax import lax
from jax.experimental import pallas as pl
from jax.experimental.pallas import tpu as pltpu
from jax.experimental.pallas import tpu_sc as plsc

_LANES = 16
_CHUNK = 128


def _split_body(K, nnsT_ref, *out_refs):
    for k in range(K):
        out_refs[k][...] = nnsT_ref[k, :]


def _make_sc_refs(B, K, NC, NS):
    NW = NC * NS
    bpw = B // NW
    mesh = plsc.VectorSubcoreMesh(core_axis_name="c", subcore_axis_name="s")

    @functools.partial(
        pl.kernel,
        mesh=mesh,
        compiler_params=pltpu.CompilerParams(use_tc_tiling_on_sc=False),
        out_type=jax.ShapeDtypeStruct((K, B), jnp.int32),
        scratch_types=[
            pltpu.VMEM((bpw,), jnp.int32),
            pltpu.VMEM((K * bpw,), jnp.int32),
            pltpu.SemaphoreType.DMA,
        ],
    )
    def sc_refs(idx_hbm, *args):
        nn_hbms = args[:K]
        refs_out = args[K]
        idx_v, refs_v, sem = args[K + 1:]
        wid = lax.axis_index("s") * NC + lax.axis_index("c")
        base = wid * bpw
        pltpu.sync_copy(idx_hbm.at[pl.ds(base, bpw)], idx_v)
        descs = []
        for k in range(K):
            for j in range(bpw // _CHUNK):
                isl = pl.ds(j * _CHUNK, _CHUNK)
                osl = pl.ds(k * bpw + j * _CHUNK, _CHUNK)
                descs.append(pltpu.async_copy(
                    nn_hbms[k].at[idx_v.at[isl]], refs_v.at[osl], sem))
        for d in descs:
            d.wait()
        for k in range(K):
            pltpu.sync_copy(refs_v.at[pl.ds(k * bpw, bpw)],
                            refs_out.at[k, pl.ds(base, bpw)])

    return sc_refs


def _make_sc_gather(B, K, D, NC, NS):
    NW = NC * NS
    bpw = B // NW
    mesh = plsc.VectorSubcoreMesh(core_axis_name="c", subcore_axis_name="s")

    HW = bpw // 2

    @functools.partial(
        pl.kernel,
        mesh=mesh,
        compiler_params=pltpu.CompilerParams(use_tc_tiling_on_sc=False),
        out_type=jax.ShapeDtypeStruct((K, B, 2 * D), jnp.float32),
        scratch_types=[
            pltpu.VMEM((K * bpw,), jnp.int32),
            pltpu.VMEM((2, bpw // 2, 2 * D), jnp.float32),
            pltpu.SemaphoreType.DMA,
            pltpu.SemaphoreType.DMA,
        ],
    )
    def sc_gather(refs_hbm, t_hbm, tg_out, refs_v, tgb, gsem, osem):
        wid = lax.axis_index("s") * NC + lax.axis_index("c")
        base = wid * bpw
        for k in range(K):
            pltpu.sync_copy(refs_hbm.at[k, pl.ds(base, bpw)],
                            refs_v.at[pl.ds(k * bpw, bpw)])
        outd = [None, None]
        for job in range(2 * K):
            k, h = job // 2, job % 2
            b = job & 1
            if outd[b] is not None:
                outd[b].wait()
            descs = []
            for j in range(HW // _CHUNK):
                off = k * bpw + h * HW + j * _CHUNK
                descs.append(pltpu.async_copy(
                    t_hbm.at[refs_v.at[pl.ds(off, _CHUNK)]],
                    tgb.at[b, pl.ds(j * _CHUNK, _CHUNK)], gsem))
            for d in descs:
                d.wait()
            outd[b] = pltpu.async_copy(
                tgb.at[b], tg_out.at[k, pl.ds(base + h * HW, HW)], osem)
        outd[0].wait()
        outd[1].wait()

    return sc_gather


def _table_body(tdT_ref, memT_ref, wpe_ref, wae_ref, bae_ref, t_ref):
    pos = lax.dot_general(tdT_ref[...], wpe_ref[...],
                          (((0,), (0,)), ((), ())),
                          preferred_element_type=jnp.float32)
    mema = lax.dot_general(memT_ref[...], wae_ref[...],
                           (((0,), (0,)), ((), ())),
                           preferred_element_type=jnp.float32)
    t = pos + mema + bae_ref[...]
    z = jnp.zeros_like(t)
    t_ref[...] = jnp.concatenate([t, z], axis=1)


def _mm(a, b, precision=None):
    return lax.dot_general(a, b, (((1,), (0,)), ((), ())),
                           preferred_element_type=jnp.float32,
                           precision=precision)


def _iota2(shape, dim):
    return lax.broadcasted_iota(jnp.int32, shape, dim)


def _combine_body(K, HID, C, FW, dT_ref, tg_ref, wq_ref, wf_ref, bf_ref,
                  out_ref):
    dT = dT_ref[...]
    pos_q2 = lax.dot_general(dT, wq_ref[...], (((0,), (0,)), ((), ())),
                             preferred_element_type=jnp.float32)
    field = lax.dot_general(dT, wf_ref[...], (((0,), (0,)), ((), ())),
                            preferred_element_type=jnp.float32) + bf_ref[...]
    diff2s = [pos_q2 - tg_ref[k] for k in range(K)]
    d2cat = jnp.concatenate(diff2s, axis=1)

    KW = K * 128
    r = _iota2((KW, K), 0)
    c = _iota2((KW, K), 1)
    sqg = ((r // 128 == c) & ((r % 128) // HID == 0)).astype(jnp.float32)
    sq = _mm(d2cat * d2cat, sqg)
    neg = -jnp.sqrt(sq)
    m = jnp.max(neg, axis=1, keepdims=True)
    es = jnp.exp(neg - m)
    dist = es / jnp.sum(es, axis=1, keepdims=True)
    c2 = _iota2((K, KW), 0)
    r2 = _iota2((K, KW), 1)
    db = (r2 // 128 == c2).astype(jnp.float32)
    distb = _mm(dist, db)
    s2 = sum(distb[:, k * 128:(k + 1) * 128] * diff2s[k] for k in range(K))

    h2 = _iota2((128, FW), 0)
    j2 = _iota2((128, FW), 1)
    hb1 = (h2 == j2 // C).astype(jnp.float32)
    j3 = _iota2((FW, C), 0)
    c3 = _iota2((FW, C), 1)
    ps1 = (c3 == j3 % C).astype(jnp.float32)
    corr = _mm(_mm(s2, hb1) * field, ps1)

    p4 = _iota2((128, C), 0)
    c4 = _iota2((128, C), 1)
    pm1 = (p4 == c4 + HID).astype(jnp.float32)
    out_ref[...] = corr - _mm(s2, pm1)


def kernel(idx, date, nns, train_dates, mem, W_pos, b_pos, W_field, b_field,
           W_adapt, b_adapt):
    B = idx.shape[0]
    K = nns.shape[1]
    N, F = train_dates.shape
    C = mem.shape[1]
    HID = W_pos.shape[1]
    D = HID + C

    tdT = train_dates.T
    memT = mem.T
    nnsT = nns.astype(jnp.int32).T
    dateT = date.T

    info = plsc.get_sparse_core_info()
    NC, NS = info.num_cores, info.num_subcores

    SB = 8192
    nn_cols = pl.pallas_call(
        functools.partial(_split_body, K),
        grid=(pl.cdiv(N, SB),),
        in_specs=[pl.BlockSpec((K, SB), lambda i: (0, i))],
        out_specs=[pl.BlockSpec((SB,), lambda i: (i,)) for _ in range(K)],
        out_shape=[jax.ShapeDtypeStruct((N,), jnp.int32) for _ in range(K)],
    )(nnsT)

    refs = _make_sc_refs(B, K, NC, NS)(idx.astype(jnp.int32), *nn_cols)

    wpe = jnp.concatenate([W_pos, jnp.zeros((F, C), jnp.float32)], axis=1)
    wae = jnp.concatenate([jnp.zeros((C, HID), jnp.float32), W_adapt], axis=1)
    bae = jnp.concatenate([jnp.zeros((HID,), jnp.float32), b_adapt]).reshape(1, D)

    NB = 4096
    tparams = pltpu.CompilerParams(fuse_transposed_lhs_in_matmul=True)
    table = pl.pallas_call(
        _table_body,
        grid=(pl.cdiv(N, NB),),
        in_specs=[
            pl.BlockSpec((F, NB), lambda i: (0, i)),
            pl.BlockSpec((C, NB), lambda i: (0, i)),
            pl.BlockSpec((F, D), lambda i: (0, 0)),
            pl.BlockSpec((C, D), lambda i: (0, 0)),
            pl.BlockSpec((1, D), lambda i: (0, 0)),
        ],
        out_specs=pl.BlockSpec((NB, 2 * D), lambda i: (i, 0)),
        out_shape=jax.ShapeDtypeStruct((N, 2 * D), jnp.float32),
        compiler_params=tparams,
    )(tdT, memT, wpe, wae, bae)

    tg = _make_sc_gather(B, K, D, NC, NS)(refs, table)

    FW = HID * C
    wq = jnp.zeros((F, 2 * D), jnp.float32).at[:, :HID].set(W_pos)

    BB = 1024
    out = pl.pallas_call(
        functools.partial(_combine_body, K, HID, C, FW),
        grid=(B // BB,),
        in_specs=[
            pl.BlockSpec((F, BB), lambda i: (0, i)),
            pl.BlockSpec((K, BB, 2 * D), lambda i: (0, i, 0)),
            pl.BlockSpec((F, 2 * D), lambda i: (0, 0)),
            pl.BlockSpec((F, FW), lambda i: (0, 0)),
            pl.BlockSpec((1, FW), lambda i: (0, 0)),
        ],
        out_specs=pl.BlockSpec((BB, C), lambda i: (i, 0)),
        out_shape=jax.ShapeDtypeStruct((B, C), jnp.float32),
        compiler_params=tparams,
    )(dateT, tg, wq, W_field, b_field.reshape(1, FW))
    return out


# speedup vs baseline: 3.3153x; 1.2161x over previous
import functools

import jax
import jax.numpy as jnp
from jax import lax
from jax.experimental import pallas as pl
from jax.experimental.pallas import tpu as pltpu
from jax.experimental.pallas import tpu_sc as plsc

_LANES = 16
_CHUNK = 128


def _split_body(K, nnsT_ref, *out_refs):
    for k in range(K):
        out_refs[k][...] = nnsT_ref[k, :]


def _make_sc_refs(B, K, NC, NS):
    NW = NC * NS
    bpw = B // NW
    mesh = plsc.VectorSubcoreMesh(core_axis_name="c", subcore_axis_name="s")

    @functools.partial(
        pl.kernel,
        mesh=mesh,
        compiler_params=pltpu.CompilerParams(use_tc_tiling_on_sc=False),
        out_type=jax.ShapeDtypeStruct((K, B), jnp.int32),
        scratch_types=[
            pltpu.VMEM((bpw,), jnp.int32),
            pltpu.VMEM((K * bpw,), jnp.int32),
            pltpu.SemaphoreType.DMA,
        ],
    )
    def sc_refs(idx_hbm, *args):
        nn_hbms = args[:K]
        refs_out = args[K]
        idx_v, refs_v, sem = args[K + 1:]
        wid = lax.axis_index("s") * NC + lax.axis_index("c")
        base = wid * bpw
        pltpu.sync_copy(idx_hbm.at[pl.ds(base, bpw)], idx_v)
        descs = []
        for k in range(K):
            for j in range(bpw // _CHUNK):
                isl = pl.ds(j * _CHUNK, _CHUNK)
                osl = pl.ds(k * bpw + j * _CHUNK, _CHUNK)
                descs.append(pltpu.async_copy(
                    nn_hbms[k].at[idx_v.at[isl]], refs_v.at[osl], sem))
        for d in descs:
            d.wait()
        for k in range(K):
            pltpu.sync_copy(refs_v.at[pl.ds(k * bpw, bpw)],
                            refs_out.at[k, pl.ds(base, bpw)])

    return sc_refs


def _make_sc_gather(B, K, D, NC, NS):
    NW = NC * NS
    bpw = B // NW
    mesh = plsc.VectorSubcoreMesh(core_axis_name="c", subcore_axis_name="s")

    HW = bpw // 2

    @functools.partial(
        pl.kernel,
        mesh=mesh,
        compiler_params=pltpu.CompilerParams(use_tc_tiling_on_sc=False),
        out_type=jax.ShapeDtypeStruct((K, B, 2 * D), jnp.float32),
        scratch_types=[
            pltpu.VMEM((K * bpw,), jnp.int32),
            pltpu.VMEM((2, bpw // 2, 2 * D), jnp.float32),
            pltpu.SemaphoreType.DMA,
            pltpu.SemaphoreType.DMA,
        ],
    )
    def sc_gather(refs_hbm, t_hbm, tg_out, refs_v, tgb, gsem, osem):
        wid = lax.axis_index("s") * NC + lax.axis_index("c")
        base = wid * bpw
        for k in range(K):
            pltpu.sync_copy(refs_hbm.at[k, pl.ds(base, bpw)],
                            refs_v.at[pl.ds(k * bpw, bpw)])
        outd = [None, None]
        for job in range(2 * K):
            k, h = job // 2, job % 2
            b = job & 1
            if outd[b] is not None:
                outd[b].wait()
            descs = []
            for j in range(HW // _CHUNK):
                off = k * bpw + h * HW + j * _CHUNK
                descs.append(pltpu.async_copy(
                    t_hbm.at[refs_v.at[pl.ds(off, _CHUNK)]],
                    tgb.at[b, pl.ds(j * _CHUNK, _CHUNK)], gsem))
            for d in descs:
                d.wait()
            outd[b] = pltpu.async_copy(
                tgb.at[b], tg_out.at[k, pl.ds(base + h * HW, HW)], osem)
        outd[0].wait()
        outd[1].wait()

    return sc_gather


def _table_body(tdT_ref, memT_ref, wpe_ref, wae_ref, bae_ref, t_ref):
    pos = lax.dot_general(tdT_ref[...], wpe_ref[...],
                          (((0,), (0,)), ((), ())),
                          preferred_element_type=jnp.float32)
    mema = lax.dot_general(memT_ref[...], wae_ref[...],
                           (((0,), (0,)), ((), ())),
                           preferred_element_type=jnp.float32)
    t = pos + mema + bae_ref[...]
    z = jnp.zeros_like(t)
    t_ref[...] = jnp.concatenate([t, z], axis=1)


def _mm(a, b, precision=None):
    return lax.dot_general(a, b, (((1,), (0,)), ((), ())),
                           preferred_element_type=jnp.float32,
                           precision=precision)


def _iota2(shape, dim):
    return lax.broadcasted_iota(jnp.int32, shape, dim)


def _combine_body(K, HID, C, FW, dT_ref, tg_ref, wq_ref, wf_ref, bf_ref,
                  out_ref):
    dT = dT_ref[...]
    pos_q2 = lax.dot_general(dT, wq_ref[...], (((0,), (0,)), ((), ())),
                             preferred_element_type=jnp.float32)
    field = lax.dot_general(dT, wf_ref[...], (((0,), (0,)), ((), ())),
                            preferred_element_type=jnp.float32) + bf_ref[...]
    diff2s = [pos_q2 - tg_ref[k] for k in range(K)]
    d2cat = jnp.concatenate(diff2s, axis=1)

    KW = K * 128
    r = _iota2((KW, K), 0)
    c = _iota2((KW, K), 1)
    sqg = ((r // 128 == c) & ((r % 128) // HID == 0)).astype(jnp.float32)
    sq = _mm(d2cat * d2cat, sqg)
    neg = -jnp.sqrt(sq)
    m = jnp.max(neg, axis=1, keepdims=True)
    es = jnp.exp(neg - m)
    dist = es / jnp.sum(es, axis=1, keepdims=True)
    c2 = _iota2((K, KW), 0)
    r2 = _iota2((K, KW), 1)
    db = (r2 // 128 == c2).astype(jnp.float32)
    distb = _mm(dist, db)
    s2 = sum(distb[:, k * 128:(k + 1) * 128] * diff2s[k] for k in range(K))

    h2 = _iota2((128, FW), 0)
    j2 = _iota2((128, FW), 1)
    hb1 = (h2 == j2 // C).astype(jnp.float32)
    j3 = _iota2((FW, C), 0)
    c3 = _iota2((FW, C), 1)
    ps1 = (c3 == j3 % C).astype(jnp.float32)
    corr = _mm(_mm(s2, hb1) * field, ps1)

    p4 = _iota2((128, C), 0)
    c4 = _iota2((128, C), 1)
    pm1 = (p4 == c4 + HID).astype(jnp.float32)
    out_ref[...] = corr - _mm(s2, pm1)


def kernel(idx, date, nns, train_dates, mem, W_pos, b_pos, W_field, b_field,
           W_adapt, b_adapt):
    B = idx.shape[0]
    K = nns.shape[1]
    N, F = train_dates.shape
    C = mem.shape[1]
    HID = W_pos.shape[1]
    D = HID + C

    tdT = train_dates.T
    memT = mem.T
    nnsT = nns.astype(jnp.int32).T
    dateT = date.T

    info = plsc.get_sparse_core_info()
    NC, NS = info.num_cores, info.num_subcores

    SB = 32768
    nn_cols = pl.pallas_call(
        functools.partial(_split_body, K),
        grid=(pl.cdiv(N, SB),),
        in_specs=[pl.BlockSpec((K, SB), lambda i: (0, i))],
        out_specs=[pl.BlockSpec((SB,), lambda i: (i,)) for _ in range(K)],
        out_shape=[jax.ShapeDtypeStruct((N,), jnp.int32) for _ in range(K)],
    )(nnsT)

    refs = _make_sc_refs(B, K, NC, NS)(idx.astype(jnp.int32), *nn_cols)

    wpe = jnp.concatenate([W_pos, jnp.zeros((F, C), jnp.float32)], axis=1)
    wae = jnp.concatenate([jnp.zeros((C, HID), jnp.float32), W_adapt], axis=1)
    bae = jnp.concatenate([jnp.zeros((HID,), jnp.float32), b_adapt]).reshape(1, D)

    NB = 8192
    tparams = pltpu.CompilerParams(fuse_transposed_lhs_in_matmul=True)
    table = pl.pallas_call(
        _table_body,
        grid=(pl.cdiv(N, NB),),
        in_specs=[
            pl.BlockSpec((F, NB), lambda i: (0, i)),
            pl.BlockSpec((C, NB), lambda i: (0, i)),
            pl.BlockSpec((F, D), lambda i: (0, 0)),
            pl.BlockSpec((C, D), lambda i: (0, 0)),
            pl.BlockSpec((1, D), lambda i: (0, 0)),
        ],
        out_specs=pl.BlockSpec((NB, 2 * D), lambda i: (i, 0)),
        out_shape=jax.ShapeDtypeStruct((N, 2 * D), jnp.float32),
        compiler_params=tparams,
    )(tdT, memT, wpe, wae, bae)

    tg = _make_sc_gather(B, K, D, NC, NS)(refs, table)

    FW = HID * C
    wq = jnp.zeros((F, 2 * D), jnp.float32).at[:, :HID].set(W_pos)

    BB = 1024
    out = pl.pallas_call(
        functools.partial(_combine_body, K, HID, C, FW),
        grid=(B // BB,),
        in_specs=[
            pl.BlockSpec((F, BB), lambda i: (0, i)),
            pl.BlockSpec((K, BB, 2 * D), lambda i: (0, i, 0)),
            pl.BlockSpec((F, 2 * D), lambda i: (0, 0)),
            pl.BlockSpec((F, FW), lambda i: (0, 0)),
            pl.BlockSpec((1, FW), lambda i: (0, 0)),
        ],
        out_specs=pl.BlockSpec((BB, C), lambda i: (i, 0)),
        out_shape=jax.ShapeDtypeStruct((B, C), jnp.float32),
        compiler_params=tparams,
    )(dateT, tg, wq, W_field, b_field.reshape(1, FW))
    return out


# speedup vs baseline: 3.6510x; 1.1013x over previous
import functools

import jax
import jax.numpy as jnp
from jax import lax
from jax.experimental import pallas as pl
from jax.experimental.pallas import tpu as pltpu
from jax.experimental.pallas import tpu_sc as plsc

_LANES = 16
_CHUNK = 128


def _split_body(K, nnsT_ref, *out_refs):
    for k in range(K):
        out_refs[k][...] = nnsT_ref[k, :]


def _make_sc_refs(B, K, NC, NS):
    NW = NC * NS
    bpw = B // NW
    mesh = plsc.VectorSubcoreMesh(core_axis_name="c", subcore_axis_name="s")

    @functools.partial(
        pl.kernel,
        mesh=mesh,
        compiler_params=pltpu.CompilerParams(use_tc_tiling_on_sc=False),
        out_type=jax.ShapeDtypeStruct((K, B), jnp.int32),
        scratch_types=[
            pltpu.VMEM((bpw,), jnp.int32),
            pltpu.VMEM((K * bpw,), jnp.int32),
            pltpu.SemaphoreType.DMA,
        ],
    )
    def sc_refs(idx_hbm, *args):
        nn_hbms = args[:K]
        refs_out = args[K]
        idx_v, refs_v, sem = args[K + 1:]
        wid = lax.axis_index("s") * NC + lax.axis_index("c")
        base = wid * bpw
        pltpu.sync_copy(idx_hbm.at[pl.ds(base, bpw)], idx_v)
        descs = []
        for k in range(K):
            for j in range(bpw // _CHUNK):
                isl = pl.ds(j * _CHUNK, _CHUNK)
                osl = pl.ds(k * bpw + j * _CHUNK, _CHUNK)
                descs.append(pltpu.async_copy(
                    nn_hbms[k].at[idx_v.at[isl]], refs_v.at[osl], sem))
        for d in descs:
            d.wait()
        for k in range(K):
            pltpu.sync_copy(refs_v.at[pl.ds(k * bpw, bpw)],
                            refs_out.at[k, pl.ds(base, bpw)])

    return sc_refs


def _make_sc_gather(B, K, D, NC, NS):
    NW = NC * NS
    bpw = B // NW
    mesh = plsc.VectorSubcoreMesh(core_axis_name="c", subcore_axis_name="s")

    HW = bpw // 2

    @functools.partial(
        pl.kernel,
        mesh=mesh,
        compiler_params=pltpu.CompilerParams(use_tc_tiling_on_sc=False),
        out_type=jax.ShapeDtypeStruct((K, B, 2 * D), jnp.float32),
        scratch_types=[
            pltpu.VMEM((K * bpw,), jnp.int32),
            pltpu.VMEM((2, bpw // 2, 2 * D), jnp.float32),
            pltpu.SemaphoreType.DMA,
            pltpu.SemaphoreType.DMA,
        ],
    )
    def sc_gather(refs_hbm, t_hbm, tg_out, refs_v, tgb, gsem, osem):
        wid = lax.axis_index("s") * NC + lax.axis_index("c")
        base = wid * bpw
        for k in range(K):
            pltpu.sync_copy(refs_hbm.at[k, pl.ds(base, bpw)],
                            refs_v.at[pl.ds(k * bpw, bpw)])
        outd = [None, None]
        for job in range(2 * K):
            k, h = job // 2, job % 2
            b = job & 1
            if outd[b] is not None:
                outd[b].wait()
            descs = []
            for j in range(HW // _CHUNK):
                off = k * bpw + h * HW + j * _CHUNK
                descs.append(pltpu.async_copy(
                    t_hbm.at[refs_v.at[pl.ds(off, _CHUNK)]],
                    tgb.at[b, pl.ds(j * _CHUNK, _CHUNK)], gsem))
            for d in descs:
                d.wait()
            outd[b] = pltpu.async_copy(
                tgb.at[b], tg_out.at[k, pl.ds(base + h * HW, HW)], osem)
        outd[0].wait()
        outd[1].wait()

    return sc_gather


def _table_body(tdT_ref, memT_ref, wpe_ref, wae_ref, bae_ref, t_ref):
    pos = lax.dot_general(tdT_ref[...], wpe_ref[...],
                          (((0,), (0,)), ((), ())),
                          preferred_element_type=jnp.float32)
    mema = lax.dot_general(memT_ref[...], wae_ref[...],
                           (((0,), (0,)), ((), ())),
                           preferred_element_type=jnp.float32)
    t = pos + mema + bae_ref[...]
    z = jnp.zeros_like(t)
    t_ref[...] = jnp.concatenate([t, z], axis=1)


def _mm(a, b, precision=None):
    return lax.dot_general(a, b, (((1,), (0,)), ((), ())),
                           preferred_element_type=jnp.float32,
                           precision=precision)


def _iota2(shape, dim):
    return lax.broadcasted_iota(jnp.int32, shape, dim)


def _combine_body(K, HID, C, FW, dT_ref, tg_ref, wq_ref, wf_ref, bf_ref,
                  out_ref):
    dT = dT_ref[...]
    pos_q2 = lax.dot_general(dT, wq_ref[...], (((0,), (0,)), ((), ())),
                             preferred_element_type=jnp.float32)
    field = lax.dot_general(dT, wf_ref[...], (((0,), (0,)), ((), ())),
                            preferred_element_type=jnp.float32) + bf_ref[...]
    diff2s = [pos_q2 - tg_ref[k] for k in range(K)]
    d2cat = jnp.concatenate(diff2s, axis=1)

    KW = K * 128
    r = _iota2((KW, K), 0)
    c = _iota2((KW, K), 1)
    sqg = ((r // 128 == c) & ((r % 128) // HID == 0)).astype(jnp.float32)
    sq = _mm(d2cat * d2cat, sqg)
    neg = -jnp.sqrt(sq)
    m = jnp.max(neg, axis=1, keepdims=True)
    es = jnp.exp(neg - m)
    dist = es / jnp.sum(es, axis=1, keepdims=True)
    c2 = _iota2((K, KW), 0)
    r2 = _iota2((K, KW), 1)
    db = (r2 // 128 == c2).astype(jnp.float32)
    distb = _mm(dist, db)
    s2 = sum(distb[:, k * 128:(k + 1) * 128] * diff2s[k] for k in range(K))

    h2 = _iota2((128, FW), 0)
    j2 = _iota2((128, FW), 1)
    hb1 = (h2 == j2 // C).astype(jnp.float32)
    j3 = _iota2((FW, C), 0)
    c3 = _iota2((FW, C), 1)
    ps1 = (c3 == j3 % C).astype(jnp.float32)
    corr = _mm(_mm(s2, hb1) * field, ps1)

    p4 = _iota2((128, C), 0)
    c4 = _iota2((128, C), 1)
    pm1 = (p4 == c4 + HID).astype(jnp.float32)
    out_ref[...] = corr - _mm(s2, pm1)


def kernel(idx, date, nns, train_dates, mem, W_pos, b_pos, W_field, b_field,
           W_adapt, b_adapt):
    B = idx.shape[0]
    K = nns.shape[1]
    N, F = train_dates.shape
    C = mem.shape[1]
    HID = W_pos.shape[1]
    D = HID + C

    tdT = train_dates.T
    memT = mem.T
    nnsT = nns.astype(jnp.int32).T
    dateT = date.T

    info = plsc.get_sparse_core_info()
    NC, NS = info.num_cores, info.num_subcores

    SB = 65536
    nn_cols = pl.pallas_call(
        functools.partial(_split_body, K),
        grid=(pl.cdiv(N, SB),),
        in_specs=[pl.BlockSpec((K, SB), lambda i: (0, i))],
        out_specs=[pl.BlockSpec((SB,), lambda i: (i,)) for _ in range(K)],
        out_shape=[jax.ShapeDtypeStruct((N,), jnp.int32) for _ in range(K)],
    )(nnsT)

    refs = _make_sc_refs(B, K, NC, NS)(idx.astype(jnp.int32), *nn_cols)

    wpe = jnp.concatenate([W_pos, jnp.zeros((F, C), jnp.float32)], axis=1)
    wae = jnp.concatenate([jnp.zeros((C, HID), jnp.float32), W_adapt], axis=1)
    bae = jnp.concatenate([jnp.zeros((HID,), jnp.float32), b_adapt]).reshape(1, D)

    NB = 16384
    tparams = pltpu.CompilerParams(fuse_transposed_lhs_in_matmul=True)
    table = pl.pallas_call(
        _table_body,
        grid=(pl.cdiv(N, NB),),
        in_specs=[
            pl.BlockSpec((F, NB), lambda i: (0, i)),
            pl.BlockSpec((C, NB), lambda i: (0, i)),
            pl.BlockSpec((F, D), lambda i: (0, 0)),
            pl.BlockSpec((C, D), lambda i: (0, 0)),
            pl.BlockSpec((1, D), lambda i: (0, 0)),
        ],
        out_specs=pl.BlockSpec((NB, 2 * D), lambda i: (i, 0)),
        out_shape=jax.ShapeDtypeStruct((N, 2 * D), jnp.float32),
        compiler_params=tparams,
    )(tdT, memT, wpe, wae, bae)

    tg = _make_sc_gather(B, K, D, NC, NS)(refs, table)

    FW = HID * C
    wq = jnp.zeros((F, 2 * D), jnp.float32).at[:, :HID].set(W_pos)

    BB = 1024
    out = pl.pallas_call(
        functools.partial(_combine_body, K, HID, C, FW),
        grid=(B // BB,),
        in_specs=[
            pl.BlockSpec((F, BB), lambda i: (0, i)),
            pl.BlockSpec((K, BB, 2 * D), lambda i: (0, i, 0)),
            pl.BlockSpec((F, 2 * D), lambda i: (0, 0)),
            pl.BlockSpec((F, FW), lambda i: (0, 0)),
            pl.BlockSpec((1, FW), lambda i: (0, 0)),
        ],
        out_specs=pl.BlockSpec((BB, C), lambda i: (i, 0)),
        out_shape=jax.ShapeDtypeStruct((B, C), jnp.float32),
        compiler_params=tparams,
    )(dateT, tg, wq, W_field, b_field.reshape(1, FW))
    return out


# speedup vs baseline: 3.6584x; 1.0020x over previous
import functools

import jax
import jax.numpy as jnp
from jax import lax
from jax.experimental import pallas as pl
from jax.experimental.pallas import tpu as pltpu
from jax.experimental.pallas import tpu_sc as plsc

_LANES = 16
_CHUNK = 128


def _split_body(K, nnsT_ref, *out_refs):
    for k in range(K):
        out_refs[k][...] = nnsT_ref[k, :]


def _make_sc_refs(B, K, NC, NS):
    NW = NC * NS
    bpw = B // NW
    mesh = plsc.VectorSubcoreMesh(core_axis_name="c", subcore_axis_name="s")

    @functools.partial(
        pl.kernel,
        mesh=mesh,
        compiler_params=pltpu.CompilerParams(use_tc_tiling_on_sc=False),
        out_type=jax.ShapeDtypeStruct((K, B), jnp.int32),
        scratch_types=[
            pltpu.VMEM((bpw,), jnp.int32),
            pltpu.VMEM((K * bpw,), jnp.int32),
            pltpu.SemaphoreType.DMA,
        ],
    )
    def sc_refs(idx_hbm, *args):
        nn_hbms = args[:K]
        refs_out = args[K]
        idx_v, refs_v, sem = args[K + 1:]
        wid = lax.axis_index("s") * NC + lax.axis_index("c")
        base = wid * bpw
        pltpu.sync_copy(idx_hbm.at[pl.ds(base, bpw)], idx_v)
        descs = []
        for k in range(K):
            for j in range(bpw // _CHUNK):
                isl = pl.ds(j * _CHUNK, _CHUNK)
                osl = pl.ds(k * bpw + j * _CHUNK, _CHUNK)
                descs.append(pltpu.async_copy(
                    nn_hbms[k].at[idx_v.at[isl]], refs_v.at[osl], sem))
        for d in descs:
            d.wait()
        for k in range(K):
            pltpu.sync_copy(refs_v.at[pl.ds(k * bpw, bpw)],
                            refs_out.at[k, pl.ds(base, bpw)])

    return sc_refs


def _make_sc_gather(B, K, D, NC, NS):
    NW = NC * NS
    bpw = B // NW
    mesh = plsc.VectorSubcoreMesh(core_axis_name="c", subcore_axis_name="s")

    HW = bpw // 2

    @functools.partial(
        pl.kernel,
        mesh=mesh,
        compiler_params=pltpu.CompilerParams(use_tc_tiling_on_sc=False),
        out_type=jax.ShapeDtypeStruct((K, B, 2 * D), jnp.float32),
        scratch_types=[
            pltpu.VMEM((K * bpw,), jnp.int32),
            pltpu.VMEM((2, bpw // 2, 2 * D), jnp.float32),
            pltpu.SemaphoreType.DMA,
            pltpu.SemaphoreType.DMA,
        ],
    )
    def sc_gather(refs_hbm, t_hbm, tg_out, refs_v, tgb, gsem, osem):
        wid = lax.axis_index("s") * NC + lax.axis_index("c")
        base = wid * bpw
        for k in range(K):
            pltpu.sync_copy(refs_hbm.at[k, pl.ds(base, bpw)],
                            refs_v.at[pl.ds(k * bpw, bpw)])
        outd = [None, None]
        for job in range(2 * K):
            k, h = job // 2, job % 2
            b = job & 1
            if outd[b] is not None:
                outd[b].wait()
            descs = []
            for j in range(HW // _CHUNK):
                off = k * bpw + h * HW + j * _CHUNK
                descs.append(pltpu.async_copy(
                    t_hbm.at[refs_v.at[pl.ds(off, _CHUNK)]],
                    tgb.at[b, pl.ds(j * _CHUNK, _CHUNK)], gsem))
            for d in descs:
                d.wait()
            outd[b] = pltpu.async_copy(
                tgb.at[b], tg_out.at[k, pl.ds(base + h * HW, HW)], osem)
        outd[0].wait()
        outd[1].wait()

    return sc_gather


def _table_body(tdT_ref, memT_ref, wpe_ref, wae_ref, bae_ref, t_ref):
    pos = lax.dot_general(tdT_ref[...], wpe_ref[...],
                          (((0,), (0,)), ((), ())),
                          preferred_element_type=jnp.float32)
    mema = lax.dot_general(memT_ref[...], wae_ref[...],
                           (((0,), (0,)), ((), ())),
                           preferred_element_type=jnp.float32)
    t = pos + mema + bae_ref[...]
    z = jnp.zeros_like(t)
    t_ref[...] = jnp.concatenate([t, z], axis=1)


def _mm(a, b, precision=None):
    return lax.dot_general(a, b, (((1,), (0,)), ((), ())),
                           preferred_element_type=jnp.float32,
                           precision=precision)


def _iota2(shape, dim):
    return lax.broadcasted_iota(jnp.int32, shape, dim)


def _combine_body(K, HID, C, FW, dT_ref, tg_ref, wq_ref, wf_ref, bf_ref,
                  out_ref):
    dT = dT_ref[...]
    pos_q2 = lax.dot_general(dT, wq_ref[...], (((0,), (0,)), ((), ())),
                             preferred_element_type=jnp.float32)
    field = lax.dot_general(dT, wf_ref[...], (((0,), (0,)), ((), ())),
                            preferred_element_type=jnp.float32) + bf_ref[...]
    diff2s = [pos_q2 - tg_ref[k] for k in range(K)]
    d2cat = jnp.concatenate(diff2s, axis=1)

    KW = K * 128
    r = _iota2((KW, K), 0)
    c = _iota2((KW, K), 1)
    sqg = ((r // 128 == c) & ((r % 128) // HID == 0)).astype(jnp.float32)
    sq = _mm(d2cat * d2cat, sqg)
    neg = -jnp.sqrt(sq)
    m = jnp.max(neg, axis=1, keepdims=True)
    es = jnp.exp(neg - m)
    dist = es / jnp.sum(es, axis=1, keepdims=True)
    c2 = _iota2((K, KW), 0)
    r2 = _iota2((K, KW), 1)
    db = (r2 // 128 == c2).astype(jnp.float32)
    distb = _mm(dist, db)
    s2 = sum(distb[:, k * 128:(k + 1) * 128] * diff2s[k] for k in range(K))

    h2 = _iota2((128, FW), 0)
    j2 = _iota2((128, FW), 1)
    hb1 = (h2 == j2 // C).astype(jnp.float32)
    j3 = _iota2((FW, C), 0)
    c3 = _iota2((FW, C), 1)
    ps1 = (c3 == j3 % C).astype(jnp.float32)
    corr = _mm(_mm(s2, hb1) * field, ps1)

    p4 = _iota2((128, C), 0)
    c4 = _iota2((128, C), 1)
    pm1 = (p4 == c4 + HID).astype(jnp.float32)
    out_ref[...] = corr - _mm(s2, pm1)


def kernel(idx, date, nns, train_dates, mem, W_pos, b_pos, W_field, b_field,
           W_adapt, b_adapt):
    B = idx.shape[0]
    K = nns.shape[1]
    N, F = train_dates.shape
    C = mem.shape[1]
    HID = W_pos.shape[1]
    D = HID + C

    tdT = train_dates.T
    memT = mem.T
    nnsT = nns.astype(jnp.int32).T
    dateT = date.T

    info = plsc.get_sparse_core_info()
    NC, NS = info.num_cores, info.num_subcores

    SB = 65536
    nn_cols = pl.pallas_call(
        functools.partial(_split_body, K),
        grid=(pl.cdiv(N, SB),),
        in_specs=[pl.BlockSpec((K, SB), lambda i: (0, i))],
        out_specs=[pl.BlockSpec((SB,), lambda i: (i,)) for _ in range(K)],
        out_shape=[jax.ShapeDtypeStruct((N,), jnp.int32) for _ in range(K)],
    )(nnsT)

    refs = _make_sc_refs(B, K, NC, NS)(idx.astype(jnp.int32), *nn_cols)

    wpe = jnp.concatenate([W_pos, jnp.zeros((F, C), jnp.float32)], axis=1)
    wae = jnp.concatenate([jnp.zeros((C, HID), jnp.float32), W_adapt], axis=1)
    bae = jnp.concatenate([jnp.zeros((HID,), jnp.float32), b_adapt]).reshape(1, D)

    NB = 16384
    tparams = pltpu.CompilerParams(fuse_transposed_lhs_in_matmul=True)
    table = pl.pallas_call(
        _table_body,
        grid=(pl.cdiv(N, NB),),
        in_specs=[
            pl.BlockSpec((F, NB), lambda i: (0, i)),
            pl.BlockSpec((C, NB), lambda i: (0, i)),
            pl.BlockSpec((F, D), lambda i: (0, 0)),
            pl.BlockSpec((C, D), lambda i: (0, 0)),
            pl.BlockSpec((1, D), lambda i: (0, 0)),
        ],
        out_specs=pl.BlockSpec((NB, 2 * D), lambda i: (i, 0)),
        out_shape=jax.ShapeDtypeStruct((N, 2 * D), jnp.float32),
        compiler_params=tparams,
    )(tdT, memT, wpe, wae, bae)

    tg = _make_sc_gather(B, K, D, NC, NS)(refs, table)

    FW = HID * C
    wq = jnp.zeros((F, 2 * D), jnp.float32).at[:, :HID].set(W_pos)

    BB = 2048
    out = pl.pallas_call(
        functools.partial(_combine_body, K, HID, C, FW),
        grid=(B // BB,),
        in_specs=[
            pl.BlockSpec((F, BB), lambda i: (0, i)),
            pl.BlockSpec((K, BB, 2 * D), lambda i: (0, i, 0)),
            pl.BlockSpec((F, 2 * D), lambda i: (0, 0)),
            pl.BlockSpec((F, FW), lambda i: (0, 0)),
            pl.BlockSpec((1, FW), lambda i: (0, 0)),
        ],
        out_specs=pl.BlockSpec((BB, C), lambda i: (i, 0)),
        out_shape=jax.ShapeDtypeStruct((B, C), jnp.float32),
        compiler_params=tparams,
    )(dateT, tg, wq, W_field, b_field.reshape(1, FW))
    return out
